# baseline (speedup 1.0000x reference)
import jax
import jax.numpy as jnp
from jax.experimental import pallas as pl
from jax.experimental.pallas import tpu as pltpu

N_EXPERTS = 64
N_ACT = 8
BLOCK_ROWS = 1024
IDX_MASK = N_EXPERTS - 1


def _gate_kernel(x_ref, wt_ref, wout_ref, iout_ref):
    scores = jnp.dot(x_ref[...], wt_ref[...], preferred_element_type=jnp.float32)

    rows = scores.shape[0]
    m = jnp.max(scores, axis=-1, keepdims=True)
    e = jnp.exp(scores - m)
    denom = jnp.sum(e, axis=-1, keepdims=True)

    col = jax.lax.broadcasted_iota(jnp.int32, (rows, N_EXPERTS), 1)
    packed = (e.view(jnp.int32) & ~IDX_MASK) | (IDX_MASK - col)

    vals = []
    cur = packed
    for _ in range(N_ACT):
        v = jnp.max(cur, axis=-1, keepdims=True)
        vals.append(v)
        cur = jnp.where(cur == v, jnp.int32(-0x80000000), cur)
    topv = jnp.concatenate(vals, axis=-1)
    idx = IDX_MASK - (topv & IDX_MASK)
    w = (topv & ~IDX_MASK).view(jnp.float32) / denom
    wout_ref[...] = w
    iout_ref[...] = idx


@jax.jit
def kernel(x, W):
    n_rows, k_dim = x.shape
    wt = W.T
    grid = (n_rows // BLOCK_ROWS,)
    weights, indices = pl.pallas_call(
        _gate_kernel,
        grid=grid,
        in_specs=[
            pl.BlockSpec((BLOCK_ROWS, k_dim), lambda i: (i, 0)),
            pl.BlockSpec((k_dim, N_EXPERTS), lambda i: (0, 0)),
        ],
        out_specs=[
            pl.BlockSpec((BLOCK_ROWS, N_ACT), lambda i: (i, 0)),
            pl.BlockSpec((BLOCK_ROWS, N_ACT), lambda i: (i, 0)),
        ],
        out_shape=[
            jax.ShapeDtypeStruct((n_rows, N_ACT), jnp.float32),
            jax.ShapeDtypeStruct((n_rows, N_ACT), jnp.int32),
        ],
    )(x, wt)
    return weights, indices


# speedup vs baseline: 1.0017x; 1.0017x over previous
import jax
import jax.numpy as jnp
from jax.experimental import pallas as pl
from jax.experimental.pallas import tpu as pltpu

N_EXPERTS = 64
N_ACT = 8
BLOCK_ROWS = 1024
IDX_MASK = N_EXPERTS - 1


ROW_CHUNK = 256


def _gate_kernel(x_ref, wt_ref, wout_ref, iout_ref):
    scores = jnp.dot(x_ref[...], wt_ref[...], preferred_element_type=jnp.float32)

    col = jax.lax.broadcasted_iota(jnp.int32, (ROW_CHUNK, N_EXPERTS), 1)
    for c in range(scores.shape[0] // ROW_CHUNK):
        s = scores[c * ROW_CHUNK : (c + 1) * ROW_CHUNK]
        m = jnp.max(s, axis=-1, keepdims=True)
        e = jnp.exp(s - m)
        denom = jnp.sum(e, axis=-1, keepdims=True)
        packed = (e.view(jnp.int32) & ~IDX_MASK) | (IDX_MASK - col)

        vals = []
        cur = packed
        for _ in range(N_ACT):
            v = jnp.max(cur, axis=-1, keepdims=True)
            vals.append(v)
            cur = jnp.where(cur == v, jnp.int32(-0x80000000), cur)
        topv = jnp.concatenate(vals, axis=-1)
        idx = IDX_MASK - (topv & IDX_MASK)
        w = (topv & ~IDX_MASK).view(jnp.float32) / denom
        wout_ref[c * ROW_CHUNK : (c + 1) * ROW_CHUNK] = w
        iout_ref[c * ROW_CHUNK : (c + 1) * ROW_CHUNK] = idx


@jax.jit
def kernel(x, W):
    n_rows, k_dim = x.shape
    wt = W.T
    grid = (n_rows // BLOCK_ROWS,)
    weights, indices = pl.pallas_call(
        _gate_kernel,
        grid=grid,
        in_specs=[
            pl.BlockSpec((BLOCK_ROWS, k_dim), lambda i: (i, 0)),
            pl.BlockSpec((k_dim, N_EXPERTS), lambda i: (0, 0)),
        ],
        out_specs=[
            pl.BlockSpec((BLOCK_ROWS, N_ACT), lambda i: (i, 0)),
            pl.BlockSpec((BLOCK_ROWS, N_ACT), lambda i: (i, 0)),
        ],
        out_shape=[
            jax.ShapeDtypeStruct((n_rows, N_ACT), jnp.float32),
            jax.ShapeDtypeStruct((n_rows, N_ACT), jnp.int32),
        ],
    )(x, wt)
    return weights, indices


# speedup vs baseline: 1.0609x; 1.0591x over previous
import jax
import jax.numpy as jnp
from jax.experimental import pallas as pl
from jax.experimental.pallas import tpu as pltpu

N_EXPERTS = 64
N_ACT = 8
BLOCK_ROWS = 1024
IDX_MASK = N_EXPERTS - 1


ROW_CHUNK = 256


def _gate_kernel(x_ref, wt_ref, wout_ref, iout_ref):
    scores = jnp.dot(x_ref[...], wt_ref[...], preferred_element_type=jnp.float32)

    col = jax.lax.broadcasted_iota(jnp.int32, (ROW_CHUNK, N_EXPERTS), 1)
    for c in range(scores.shape[0] // ROW_CHUNK):
        s = scores[c * ROW_CHUNK : (c + 1) * ROW_CHUNK]
        m = jnp.max(s, axis=-1, keepdims=True)
        e = jnp.exp(s - m)
        denom = jnp.sum(e, axis=-1, keepdims=True)
        packed = ((e.view(jnp.int32) & ~IDX_MASK) | (IDX_MASK - col)).view(
            jnp.float32
        )

        vals = []
        cur = packed
        for _ in range(N_ACT):
            v = jnp.max(cur, axis=-1, keepdims=True)
            vals.append(v)
            cur = jnp.where(cur == v, jnp.float32(0.0), cur)
        topv = jnp.concatenate(vals, axis=-1).view(jnp.int32)
        idx = IDX_MASK - (topv & IDX_MASK)
        w = (topv & ~IDX_MASK).view(jnp.float32) / denom
        wout_ref[c * ROW_CHUNK : (c + 1) * ROW_CHUNK] = w
        iout_ref[c * ROW_CHUNK : (c + 1) * ROW_CHUNK] = idx


@jax.jit
def kernel(x, W):
    n_rows, k_dim = x.shape
    wt = W.T
    grid = (n_rows // BLOCK_ROWS,)
    weights, indices = pl.pallas_call(
        _gate_kernel,
        grid=grid,
        in_specs=[
            pl.BlockSpec((BLOCK_ROWS, k_dim), lambda i: (i, 0)),
            pl.BlockSpec((k_dim, N_EXPERTS), lambda i: (0, 0)),
        ],
        out_specs=[
            pl.BlockSpec((BLOCK_ROWS, N_ACT), lambda i: (i, 0)),
            pl.BlockSpec((BLOCK_ROWS, N_ACT), lambda i: (i, 0)),
        ],
        out_shape=[
            jax.ShapeDtypeStruct((n_rows, N_ACT), jnp.float32),
            jax.ShapeDtypeStruct((n_rows, N_ACT), jnp.int32),
        ],
    )(x, wt)
    return weights, indices


# speedup vs baseline: 1.0689x; 1.0075x over previous
import jax
import jax.numpy as jnp
from jax.experimental import pallas as pl
from jax.experimental.pallas import tpu as pltpu

N_EXPERTS = 64
N_ACT = 8
BLOCK_ROWS = 1024
IDX_MASK = N_EXPERTS - 1


ROW_CHUNK = 256


def _gate_kernel(x_ref, wt_ref, wout_ref, iout_ref):
    col = jax.lax.broadcasted_iota(jnp.int32, (ROW_CHUNK, N_EXPERTS), 1)
    for c in range(x_ref.shape[0] // ROW_CHUNK):
        s = jnp.dot(
            x_ref[c * ROW_CHUNK : (c + 1) * ROW_CHUNK],
            wt_ref[...],
            preferred_element_type=jnp.float32,
        )
        m = jnp.max(s, axis=-1, keepdims=True)
        e = jnp.exp(s - m)
        denom = jnp.sum(e, axis=-1, keepdims=True)
        packed = ((e.view(jnp.int32) & ~IDX_MASK) | (IDX_MASK - col)).view(
            jnp.float32
        )

        vals = []
        cur = packed
        for _ in range(N_ACT):
            v = jnp.max(cur, axis=-1, keepdims=True)
            vals.append(v)
            cur = jnp.where(cur == v, jnp.float32(0.0), cur)
        topv = jnp.concatenate(vals, axis=-1).view(jnp.int32)
        idx = IDX_MASK - (topv & IDX_MASK)
        w = (topv & ~IDX_MASK).view(jnp.float32) / denom
        wout_ref[c * ROW_CHUNK : (c + 1) * ROW_CHUNK] = w
        iout_ref[c * ROW_CHUNK : (c + 1) * ROW_CHUNK] = idx


@jax.jit
def kernel(x, W):
    n_rows, k_dim = x.shape
    wt = W.T
    grid = (n_rows // BLOCK_ROWS,)
    weights, indices = pl.pallas_call(
        _gate_kernel,
        grid=grid,
        in_specs=[
            pl.BlockSpec((BLOCK_ROWS, k_dim), lambda i: (i, 0)),
            pl.BlockSpec((k_dim, N_EXPERTS), lambda i: (0, 0)),
        ],
        out_specs=[
            pl.BlockSpec((BLOCK_ROWS, N_ACT), lambda i: (i, 0)),
            pl.BlockSpec((BLOCK_ROWS, N_ACT), lambda i: (i, 0)),
        ],
        out_shape=[
            jax.ShapeDtypeStruct((n_rows, N_ACT), jnp.float32),
            jax.ShapeDtypeStruct((n_rows, N_ACT), jnp.int32),
        ],
    )(x, wt)
    return weights, indices
